# Initial kernel scaffold; baseline (speedup 1.0000x reference)
import jax
import jax.numpy as jnp
from jax.experimental import pallas as pl


def kernel(x, edge_index, edge_attr, group_emb, W1, b1, gamma, beta, We, be, W2, b2, W3, b3):
    raise NotImplementedError("write your pallas kernel here")


# speedup vs baseline: 3.1223x; 3.1223x over previous
import functools

import jax
import jax.numpy as jnp
from jax import lax
from jax.experimental import pallas as pl
from jax.experimental.pallas import tpu as pltpu
from jax.experimental.pallas import tpu_sc as plsc

_N = 10000
_E = 320000
_H = 128
_ED = 16
_NG = 8
_NC = 2
_NS = 16
_NW = _NC * _NS
_CH = 128
_NCHUNK = _E // _CH
_PER_W = _NCHUNK // _NW
_EXTRA = _NCHUNK - _PER_W * _NW
_RPT = 624
_REM = _N - _RPT * _NS


def _encode_body(x_ref, w1a_ref, w1b_ref, gemb_ref, b1_ref, gamma_ref, beta_ref, h_ref):
    xw = jnp.dot(x_ref[...], w1a_ref[...], preferred_element_type=jnp.float32)
    row = lax.broadcasted_iota(jnp.int32, (_N, _NG), 0)
    gid = jnp.where(row < 32, row // 4, 0)
    lane = lax.broadcasted_iota(jnp.int32, (_N, _NG), 1)
    onehot = (gid == lane).astype(jnp.float32)
    gw = jnp.dot(gemb_ref[...], w1b_ref[...], preferred_element_type=jnp.float32)
    encw = jnp.dot(onehot, gw, preferred_element_type=jnp.float32)
    h1 = jnp.maximum(xw + encw + b1_ref[...], 0.0)
    m = jnp.mean(h1, axis=0, keepdims=True)
    v = jnp.mean((h1 - m) ** 2, axis=0, keepdims=True)
    h_ref[...] = (h1 - m) * lax.rsqrt(v + 1e-5) * gamma_ref[...] + beta_ref[...]


def _encode(x, w1a, w1b, gemb, b1, gamma, beta):
    return pl.pallas_call(
        _encode_body,
        out_shape=jax.ShapeDtypeStruct((_N, _H), jnp.float32),
    )(x, w1a, w1b, gemb, b1, gamma, beta)


_EB = 8000


def _edge_mlp_body(ea_ref, we_ref, be_ref, t_ref):
    t_ref[...] = (
        jnp.dot(ea_ref[...], we_ref[...], preferred_element_type=jnp.float32)
        + be_ref[...]
    )


def _edge_mlp(edge_attr, we, be):
    grid = _E // _EB
    return pl.pallas_call(
        _edge_mlp_body,
        grid=(grid,),
        in_specs=[
            pl.BlockSpec((_EB, _ED), lambda i: (i, 0)),
            pl.BlockSpec((_ED, _H), lambda i: (0, 0)),
            pl.BlockSpec((1, _H), lambda i: (0, 0)),
        ],
        out_specs=pl.BlockSpec((_EB, _H), lambda i: (i, 0)),
        out_shape=jax.ShapeDtypeStruct((_E, _H), jnp.float32),
    )(edge_attr, we, be)


def _sc_agg_body(h_hbm, t_hbm, src_hbm, dst_hbm, z_hbm, out_hbm,
                 sidx, didx, tbuf, rbuf, agg_s, sem_t, sem_r):
    c = lax.axis_index("c")
    s = lax.axis_index("s")
    w = c * _NS + s

    pltpu.sync_copy(z_hbm, agg_s.at[pl.ds(s * _RPT, _RPT)])

    @pl.when(s == _NS - 1)
    def _zero_tail():
        pltpu.sync_copy(z_hbm.at[pl.ds(0, _REM)],
                        agg_s.at[pl.ds(_NS * _RPT, _REM)])

    plsc.subcore_barrier()

    nk = jnp.where(w < _EXTRA, _PER_W + 1, _PER_W)

    def chunk(k, carry):
        base = (w + k * _NW) * _CH
        pltpu.sync_copy(src_hbm.at[pl.ds(base, _CH)], sidx)
        pltpu.sync_copy(dst_hbm.at[pl.ds(base, _CH)], didx)
        cp_t = pltpu.async_copy(t_hbm.at[pl.ds(base, _CH)], tbuf, sem_t)
        cp_r = pltpu.async_copy(h_hbm.at[sidx], rbuf, sem_r)
        cp_t.wait()
        cp_r.wait()

        def row(r, carry2):
            for j in range(_H // 16):
                sl = pl.ds(j * 16, 16)
                tbuf[r, sl] = jnp.maximum(tbuf[r, sl] + rbuf[r, sl], 0.0)
            return carry2

        lax.fori_loop(0, _CH, row, 0)
        pltpu.sync_copy(tbuf, agg_s.at[didx], add=True)
        return carry

    lax.fori_loop(0, nk, chunk, 0)
    plsc.subcore_barrier()
    pltpu.sync_copy(agg_s.at[pl.ds(s * _RPT, _RPT)],
                    out_hbm.at[pl.ds(c * _N + s * _RPT, _RPT)])

    @pl.when(s == _NS - 1)
    def _write_tail():
        pltpu.sync_copy(agg_s.at[pl.ds(_NS * _RPT, _REM)],
                        out_hbm.at[pl.ds(c * _N + _NS * _RPT, _REM)])


def _sc_aggregate(h, t, src, dst):
    mesh = plsc.VectorSubcoreMesh(
        core_axis_name="c", subcore_axis_name="s",
        num_cores=_NC, num_subcores=_NS)
    zrows = jnp.zeros((_RPT, _H), jnp.float32)
    fn = pl.kernel(
        _sc_agg_body,
        out_type=jax.ShapeDtypeStruct((_NC * _N, _H), jnp.float32),
        mesh=mesh,
        scratch_types=[
            pltpu.VMEM((_CH,), jnp.int32),
            pltpu.VMEM((_CH,), jnp.int32),
            pltpu.VMEM((_CH, _H), jnp.float32),
            pltpu.VMEM((_CH, _H), jnp.float32),
            pltpu.VMEM_SHARED((_N, _H), jnp.float32),
            pltpu.SemaphoreType.DMA,
            pltpu.SemaphoreType.DMA,
        ],
    )
    return fn(h, t, src, dst, zrows)


_NB = 2000


def _finish_body(h_ref, a0_ref, a1_ref, w2_ref, b2_ref, w3_ref, b3_ref, o_ref):
    a = h_ref[...] + a0_ref[...] + a1_ref[...]
    u = jnp.maximum(jnp.dot(a, w2_ref[...], preferred_element_type=jnp.float32)
                    + b2_ref[...], 0.0)
    o = jnp.dot(u, w3_ref[...], preferred_element_type=jnp.float32) + b3_ref[...]
    o_ref[...] = jnp.maximum(o, 0.0)


def _finish(h, a0, a1, w2, b2, w3, b3):
    grid = _N // _NB
    node_spec = pl.BlockSpec((_NB, _H), lambda i: (i, 0))
    mat_spec = pl.BlockSpec((_H, _H), lambda i: (0, 0))
    vec_spec = pl.BlockSpec((1, _H), lambda i: (0, 0))
    return pl.pallas_call(
        _finish_body,
        grid=(grid,),
        in_specs=[node_spec, node_spec, node_spec, mat_spec, vec_spec, mat_spec, vec_spec],
        out_specs=node_spec,
        out_shape=jax.ShapeDtypeStruct((_N, _H), jnp.float32),
    )(h, a0, a1, w2, b2, w3, b3)


def kernel(x, edge_index, edge_attr, group_emb, W1, b1, gamma, beta, We, be, W2, b2, W3, b3):
    ei = edge_index.astype(jnp.int32)
    src = ei[0]
    dst = ei[1]
    w1a = W1[:_H]
    w1b = W1[_H:]
    h = _encode(x, w1a, w1b, group_emb, b1.reshape(1, _H),
                gamma.reshape(1, _H), beta.reshape(1, _H))
    t = _edge_mlp(edge_attr, We, be.reshape(1, _H))
    agg = _sc_aggregate(h, t, src, dst)
    out = _finish(h, agg[:_N], agg[_N:], W2, b2.reshape(1, _H), W3, b3.reshape(1, _H))
    return (out, edge_attr)


# speedup vs baseline: 3.5839x; 1.1479x over previous
import functools

import jax
import jax.numpy as jnp
from jax import lax
from jax.experimental import pallas as pl
from jax.experimental.pallas import tpu as pltpu
from jax.experimental.pallas import tpu_sc as plsc

_N = 10000
_E = 320000
_H = 128
_ED = 16
_NG = 8
_NC = 2
_NS = 16
_NW = _NC * _NS
_CH = 64
_NCHUNK = _E // _CH
_PER_W = _NCHUNK // _NW
_TAILW = _NCHUNK - _PER_W * _NW
_MAIN_E = _PER_W * _CH
_RPT = 624
_REM = _N - _RPT * _NS


def _encode_body(x_ref, w1a_ref, w1b_ref, gemb_ref, b1_ref, gamma_ref, beta_ref, h_ref):
    xw = jnp.dot(x_ref[...], w1a_ref[...], preferred_element_type=jnp.float32)
    row = lax.broadcasted_iota(jnp.int32, (_N, _NG), 0)
    gid = jnp.where(row < 32, row // 4, 0)
    lane = lax.broadcasted_iota(jnp.int32, (_N, _NG), 1)
    onehot = (gid == lane).astype(jnp.float32)
    gw = jnp.dot(gemb_ref[...], w1b_ref[...], preferred_element_type=jnp.float32)
    encw = jnp.dot(onehot, gw, preferred_element_type=jnp.float32)
    h1 = jnp.maximum(xw + encw + b1_ref[...], 0.0)
    m = jnp.mean(h1, axis=0, keepdims=True)
    v = jnp.mean((h1 - m) ** 2, axis=0, keepdims=True)
    h_ref[...] = (h1 - m) * lax.rsqrt(v + 1e-5) * gamma_ref[...] + beta_ref[...]


def _encode(x, w1a, w1b, gemb, b1, gamma, beta):
    return pl.pallas_call(
        _encode_body,
        out_shape=jax.ShapeDtypeStruct((_N, _H), jnp.float32),
    )(x, w1a, w1b, gemb, b1, gamma, beta)


_EB = 8000


def _edge_mlp_body(ea_ref, we_ref, be_ref, t_ref):
    t_ref[...] = (
        jnp.dot(ea_ref[...], we_ref[...], preferred_element_type=jnp.float32)
        + be_ref[...]
    )


def _edge_mlp(edge_attr, we, be):
    grid = _E // _EB
    return pl.pallas_call(
        _edge_mlp_body,
        grid=(grid,),
        in_specs=[
            pl.BlockSpec((_EB, _ED), lambda i: (i, 0)),
            pl.BlockSpec((_ED, _H), lambda i: (0, 0)),
            pl.BlockSpec((1, _H), lambda i: (0, 0)),
        ],
        out_specs=pl.BlockSpec((_EB, _H), lambda i: (i, 0)),
        out_shape=jax.ShapeDtypeStruct((_E, _H), jnp.float32),
    )(edge_attr, we, be)


def _sc_agg_body(h_hbm, t_hbm, src_hbm, dst_hbm, z_hbm, out_hbm,
                 sidx0, sidx1, sidx2, sidx3, didx0, didx1, didx2, didx3,
                 tbuf0, tbuf1, rbuf0, rbuf1, agg_s,
                 sem_i0, sem_i1, sem_i2, sem_i3,
                 sem_t0, sem_t1, sem_r0, sem_r1):
    c = lax.axis_index("c")
    s = lax.axis_index("s")
    w = c * _NS + s
    sidx = (sidx0, sidx1, sidx2, sidx3)
    didx = (didx0, didx1, didx2, didx3)
    tbuf = (tbuf0, tbuf1)
    rbuf = (rbuf0, rbuf1)
    sem_i = (sem_i0, sem_i1, sem_i2, sem_i3)
    sem_t = (sem_t0, sem_t1)
    sem_r = (sem_r0, sem_r1)
    ebase = w * _MAIN_E

    pltpu.sync_copy(z_hbm, agg_s.at[pl.ds(s * _RPT, _RPT)])

    @pl.when(s == _NS - 1)
    def _zero_tail():
        pltpu.sync_copy(z_hbm.at[pl.ds(0, _REM)],
                        agg_s.at[pl.ds(_NS * _RPT, _REM)])

    plsc.subcore_barrier()

    def issue_idx(base, r4):
        pltpu.async_copy(src_hbm.at[pl.ds(base, _CH)], sidx[r4], sem_i[r4])
        pltpu.async_copy(dst_hbm.at[pl.ds(base, _CH)], didx[r4], sem_i[r4])

    def wait_idx(r4):
        pltpu.make_async_copy(src_hbm.at[pl.ds(0, _CH)], sidx[r4], sem_i[r4]).wait()
        pltpu.make_async_copy(dst_hbm.at[pl.ds(0, _CH)], didx[r4], sem_i[r4]).wait()

    def issue_loads(tb, r4, b):
        pltpu.async_copy(t_hbm.at[pl.ds(tb, _CH)], tbuf[b], sem_t[b])
        pltpu.async_copy(h_hbm.at[sidx[r4]], rbuf[b], sem_r[b])

    def wait_loads(r4, b):
        pltpu.make_async_copy(t_hbm.at[pl.ds(0, _CH)], tbuf[b], sem_t[b]).wait()
        pltpu.make_async_copy(h_hbm.at[sidx[r4]], rbuf[b], sem_r[b]).wait()

    def compute(b):
        tb_, rb_ = tbuf[b], rbuf[b]

        def row(r, carry):
            for j in range(_H // 16):
                sl = pl.ds(j * 16, 16)
                rb_[r, sl] = jnp.maximum(tb_[r, sl] + rb_[r, sl], 0.0)
            return carry
        lax.fori_loop(0, _CH, row, 0)

    issue_idx(ebase, 0)
    issue_idx(ebase + _CH, 1)
    wait_idx(0)
    issue_loads(ebase, 0, 0)

    def quad(q, carry):
        for j4 in range(4):
            k = 4 * q + j4
            b = j4 & 1

            wait_loads(j4, b)

            @pl.when(k + 2 < _PER_W)
            def _():
                issue_idx(ebase + (k + 2) * _CH, (j4 + 2) % 4)

            compute(b)

            @pl.when(k + 1 < _PER_W)
            def _():
                wait_idx((j4 + 1) % 4)
                issue_loads(ebase + (k + 1) * _CH, (j4 + 1) % 4, 1 - b)

            pltpu.sync_copy(rbuf[b], agg_s.at[didx[j4]], add=True)
        return carry

    lax.fori_loop(0, _PER_W // 4, quad, 0)

    @pl.when(w < _TAILW)
    def _tail():
        tb = _NW * _MAIN_E + w * _CH
        issue_idx(tb, 0)
        wait_idx(0)
        issue_loads(tb, 0, 0)
        wait_loads(0, 0)
        compute(0)
        pltpu.sync_copy(rbuf[0], agg_s.at[didx[0]], add=True)

    plsc.subcore_barrier()

    pltpu.sync_copy(agg_s.at[pl.ds(s * _RPT, _RPT)],
                    out_hbm.at[pl.ds(c * _N + s * _RPT, _RPT)])

    @pl.when(s == _NS - 1)
    def _write_tail():
        pltpu.sync_copy(agg_s.at[pl.ds(_NS * _RPT, _REM)],
                        out_hbm.at[pl.ds(c * _N + _NS * _RPT, _REM)])


def _sc_aggregate(h, t, src, dst):
    mesh = plsc.VectorSubcoreMesh(
        core_axis_name="c", subcore_axis_name="s",
        num_cores=_NC, num_subcores=_NS)
    zrows = jnp.zeros((_RPT, _H), jnp.float32)
    fn = pl.kernel(
        _sc_agg_body,
        out_type=jax.ShapeDtypeStruct((_NC * _N, _H), jnp.float32),
        mesh=mesh,
        scratch_types=[pltpu.VMEM((_CH,), jnp.int32)] * 8 + [
            pltpu.VMEM((_CH, _H), jnp.float32),
            pltpu.VMEM((_CH, _H), jnp.float32),
            pltpu.VMEM((_CH, _H), jnp.float32),
            pltpu.VMEM((_CH, _H), jnp.float32),
            pltpu.VMEM_SHARED((_N, _H), jnp.float32),
        ] + [pltpu.SemaphoreType.DMA] * 8,
    )
    return fn(h, t, src, dst, zrows)


_NB = 2000


def _finish_body(h_ref, a0_ref, a1_ref, w2_ref, b2_ref, w3_ref, b3_ref, o_ref):
    a = h_ref[...] + a0_ref[...] + a1_ref[...]
    u = jnp.maximum(jnp.dot(a, w2_ref[...], preferred_element_type=jnp.float32)
                    + b2_ref[...], 0.0)
    o = jnp.dot(u, w3_ref[...], preferred_element_type=jnp.float32) + b3_ref[...]
    o_ref[...] = jnp.maximum(o, 0.0)


def _finish(h, a0, a1, w2, b2, w3, b3):
    grid = _N // _NB
    node_spec = pl.BlockSpec((_NB, _H), lambda i: (i, 0))
    mat_spec = pl.BlockSpec((_H, _H), lambda i: (0, 0))
    vec_spec = pl.BlockSpec((1, _H), lambda i: (0, 0))
    return pl.pallas_call(
        _finish_body,
        grid=(grid,),
        in_specs=[node_spec, node_spec, node_spec, mat_spec, vec_spec, mat_spec, vec_spec],
        out_specs=node_spec,
        out_shape=jax.ShapeDtypeStruct((_N, _H), jnp.float32),
    )(h, a0, a1, w2, b2, w3, b3)


def kernel(x, edge_index, edge_attr, group_emb, W1, b1, gamma, beta, We, be, W2, b2, W3, b3):
    ei = edge_index.astype(jnp.int32)
    src = ei[0]
    dst = ei[1]
    w1a = W1[:_H]
    w1b = W1[_H:]
    h = _encode(x, w1a, w1b, group_emb, b1.reshape(1, _H),
                gamma.reshape(1, _H), beta.reshape(1, _H))
    t = _edge_mlp(edge_attr, We, be.reshape(1, _H))
    agg = _sc_aggregate(h, t, src, dst)
    out = _finish(h, agg[:_N], agg[_N:], W2, b2.reshape(1, _H), W3, b3.reshape(1, _H))
    return (out, edge_attr)
